# Initial kernel scaffold; baseline (speedup 1.0000x reference)
import functools

import jax
import jax.numpy as jnp
from jax import lax
from jax.experimental import pallas as pl
from jax.experimental.pallas import tpu as pltpu
from jax.experimental.pallas import tpu_sc as plsc

N = 10000
NPAD = 10240
D_IN = 256
H_DIM = 64
C_DIM = 64
K_HOPS = 10

NC = 2
NS = 16
NW = NC * NS
CH = 128
NCH = 42
EPW = NCH * CH
E_TOT = NW * EPW
ZR = NPAD // NS

_mesh = plsc.VectorSubcoreMesh(core_axis_name="c", subcore_axis_name="s")


@jax.jit
def _sc_degree(dst3, onehot):
    @functools.partial(
        pl.kernel,
        out_type=jax.ShapeDtypeStruct((NC, NPAD, 16), jnp.float32),
        mesh=_mesh,
        scratch_types=[
            pltpu.VMEM((NCH, CH), jnp.int32),
            pltpu.VMEM((CH, 16), jnp.float32),
            pltpu.VMEM((ZR, 16), jnp.float32),
            pltpu.VMEM_SHARED((NPAD, 16), jnp.float32),
        ],
    )
    def body(dst_hbm, oh_hbm, out_hbm, idx_v, ones_v, zero_v, acc_sh):
        c = lax.axis_index("c")
        s = lax.axis_index("s")
        w = c * NS + s
        @pl.loop(0, ZR)
        def _(i):
            zero_v[i, pl.ds(0, 16)] = jnp.zeros((16,), jnp.float32)

        pltpu.sync_copy(zero_v, acc_sh.at[pl.ds(s * ZR, ZR)])
        pltpu.sync_copy(dst_hbm.at[w], idx_v)
        pltpu.sync_copy(oh_hbm, ones_v)
        plsc.subcore_barrier()

        @pl.loop(0, NCH)
        def _(j):
            pltpu.sync_copy(ones_v, acc_sh.at[idx_v.at[j]], add=True)

        plsc.subcore_barrier()
        pltpu.sync_copy(acc_sh.at[pl.ds(s * ZR, ZR)],
                        out_hbm.at[c, pl.ds(s * ZR, ZR)])

    return body(dst3, onehot)


@jax.jit
def _sc_step(z, src3, dst3):
    @functools.partial(
        pl.kernel,
        out_type=jax.ShapeDtypeStruct((NC, NPAD, C_DIM), jnp.float32),
        mesh=_mesh,
        scratch_types=[
            pltpu.VMEM((NCH, CH), jnp.int32),
            pltpu.VMEM((NCH, CH), jnp.int32),
            pltpu.VMEM((CH, C_DIM), jnp.float32),
            pltpu.VMEM_SHARED((NPAD, C_DIM), jnp.float32),
            pltpu.SemaphoreType.DMA,
        ],
    )
    def body(z_hbm, src_hbm, dst_hbm, out_hbm, src_v, dst_v, rows_v, acc_sh,
             sem):
        c = lax.axis_index("c")
        s = lax.axis_index("s")
        w = c * NS + s
        init = pltpu.async_copy(z_hbm.at[pl.ds(s * ZR, ZR)],
                                acc_sh.at[pl.ds(s * ZR, ZR)], sem)
        pltpu.sync_copy(src_hbm.at[w], src_v)
        pltpu.sync_copy(dst_hbm.at[w], dst_v)
        init.wait()
        plsc.subcore_barrier()

        @pl.loop(0, NCH)
        def _(j):
            pltpu.async_copy(z_hbm.at[src_v.at[j]], rows_v, sem).wait()
            pltpu.sync_copy(rows_v, acc_sh.at[dst_v.at[j]], add=True)

        plsc.subcore_barrier()
        pltpu.sync_copy(acc_sh.at[pl.ds(s * ZR, ZR)],
                        out_hbm.at[c, pl.ds(s * ZR, ZR)])

    return body(z, src3, dst3)


def _mlp_body(x_ref, w1_ref, b1_ref, w2_ref, b2_ref, h_ref):
    a = jnp.dot(x_ref[...], w1_ref[...],
                preferred_element_type=jnp.float32,
                precision=lax.Precision.HIGHEST)
    a = jnp.maximum(a + b1_ref[...], 0.0)
    h_ref[...] = jnp.dot(a, w2_ref[...],
                         preferred_element_type=jnp.float32,
                         precision=lax.Precision.HIGHEST) + b2_ref[...]


@jax.jit
def _tc_mlp(xp, W1, b1, W2, b2):
    blk = 1024
    return pl.pallas_call(
        _mlp_body,
        grid=(NPAD // blk,),
        in_specs=[
            pl.BlockSpec((blk, D_IN), lambda i: (i, 0)),
            pl.BlockSpec((D_IN, H_DIM), lambda i: (0, 0)),
            pl.BlockSpec((1, H_DIM), lambda i: (0, 0)),
            pl.BlockSpec((H_DIM, C_DIM), lambda i: (0, 0)),
            pl.BlockSpec((1, C_DIM), lambda i: (0, 0)),
        ],
        out_specs=pl.BlockSpec((blk, C_DIM), lambda i: (i, 0)),
        out_shape=jax.ShapeDtypeStruct((NPAD, C_DIM), jnp.float32),
    )(xp, W1, b1.reshape(1, H_DIM), W2, b2.reshape(1, C_DIM))


def _prep_body(hacc_ref, h_ref, t_ref, dinv_ref, dinv2_ref, z_ref, zacc_ref):
    deg = hacc_ref[0, :, 0:1] + hacc_ref[1, :, 0:1] + 1.0
    dinv = lax.rsqrt(deg)
    z = dinv * h_ref[...]
    dinv_ref[...] = dinv
    dinv2_ref[...] = dinv * dinv
    z_ref[...] = z
    zacc_ref[...] = t_ref[0, 0] * z


@jax.jit
def _tc_prep(hacc, h, t0):
    return pl.pallas_call(
        _prep_body,
        out_shape=(
            jax.ShapeDtypeStruct((NPAD, 1), jnp.float32),
            jax.ShapeDtypeStruct((NPAD, 1), jnp.float32),
            jax.ShapeDtypeStruct((NPAD, C_DIM), jnp.float32),
            jax.ShapeDtypeStruct((NPAD, C_DIM), jnp.float32),
        ),
    )(hacc, h, t0)


def _rescale_body(acc_ref, z_ref, zacc_ref, dinv2_ref, t_ref, znew_ref,
                  zaccnew_ref):
    sz = acc_ref[0] + acc_ref[1] - z_ref[...]
    znew = dinv2_ref[...] * sz
    znew_ref[...] = znew
    zaccnew_ref[...] = zacc_ref[...] + t_ref[0, 0] * znew


@jax.jit
def _tc_rescale(acc, z, zacc, dinv2, tk):
    return pl.pallas_call(
        _rescale_body,
        out_shape=(
            jax.ShapeDtypeStruct((NPAD, C_DIM), jnp.float32),
            jax.ShapeDtypeStruct((NPAD, C_DIM), jnp.float32),
        ),
    )(acc, z, zacc, dinv2, tk)


def _final_body(zacc_ref, dinv_ref, out_ref):
    hid = zacc_ref[...] / dinv_ref[...]
    m = jnp.max(hid, axis=1, keepdims=True)
    e = jnp.exp(hid - m)
    lse = jnp.log(jnp.sum(e, axis=1, keepdims=True))
    out_ref[...] = hid - m - lse


@jax.jit
def _tc_final(zacc, dinv):
    return pl.pallas_call(
        _final_body,
        out_shape=jax.ShapeDtypeStruct((NPAD, C_DIM), jnp.float32),
    )(zacc, dinv)


def kernel(x, edge_index, W1, b1, W2, b2, temp):
    E = edge_index.shape[1]
    pad = E_TOT - E
    pad_idx = N + (jnp.arange(pad, dtype=jnp.int32) % (NPAD - N))
    src3 = jnp.concatenate([edge_index[0], pad_idx]).reshape(NW, NCH, CH)
    dst3 = jnp.concatenate([edge_index[1], pad_idx]).reshape(NW, NCH, CH)
    onehot = jnp.zeros((CH, 16), jnp.float32).at[:, 0].set(1.0)
    xp = jnp.pad(x, ((0, NPAD - N), (0, 0)))

    hacc = _sc_degree(dst3, onehot)
    h = _tc_mlp(xp, W1, b1, W2, b2)
    tc = temp.reshape(1, K_HOPS + 1)
    dinv, dinv2, z, zacc = _tc_prep(hacc, h, tc[:, 0:1])
    for k in range(K_HOPS):
        acc = _sc_step(z, src3, dst3)
        z, zacc = _tc_rescale(acc, z, zacc, dinv2, tc[:, k + 1:k + 2])
    out = _tc_final(zacc, dinv)
    return out[:N]


# speedup vs baseline: 13.3053x; 13.3053x over previous
import functools

import jax
import jax.numpy as jnp
from jax import lax
from jax.experimental import pallas as pl
from jax.experimental.pallas import tpu as pltpu
from jax.experimental.pallas import tpu_sc as plsc

N = 10000
NPAD = 10240
D_IN = 256
H_DIM = 64
C_DIM = 64
K_HOPS = 10

NC = 2
NS = 16
NW = NC * NS
CH = 128
NCH = 42
EPW = NCH * CH
E_TOT = NW * EPW
ZR = NPAD // NS

_mesh = plsc.VectorSubcoreMesh(core_axis_name="c", subcore_axis_name="s")
_sc_params = pltpu.CompilerParams(use_tc_tiling_on_sc=False)


@jax.jit
def _sc_degree(dst3, onehot):
    @functools.partial(
        pl.kernel,
        out_type=jax.ShapeDtypeStruct((NC, NPAD, 16), jnp.float32),
        mesh=_mesh,
        scratch_types=[
            pltpu.VMEM((NCH, CH), jnp.int32),
            pltpu.VMEM((CH, 16), jnp.float32),
            pltpu.VMEM((ZR, 16), jnp.float32),
            pltpu.VMEM_SHARED((NPAD, 16), jnp.float32),
        ],
        compiler_params=_sc_params,
    )
    def body(dst_hbm, oh_hbm, out_hbm, idx_v, ones_v, zero_v, acc_sh):
        c = lax.axis_index("c")
        s = lax.axis_index("s")
        w = c * NS + s
        @pl.loop(0, ZR)
        def _(i):
            zero_v[i, pl.ds(0, 16)] = jnp.zeros((16,), jnp.float32)

        pltpu.sync_copy(zero_v, acc_sh.at[pl.ds(s * ZR, ZR)])
        pltpu.sync_copy(dst_hbm.at[w], idx_v)
        pltpu.sync_copy(oh_hbm, ones_v)
        plsc.subcore_barrier()

        @pl.loop(0, NCH)
        def _(j):
            pltpu.sync_copy(ones_v, acc_sh.at[idx_v.at[j]], add=True)

        plsc.subcore_barrier()
        pltpu.sync_copy(acc_sh.at[pl.ds(s * ZR, ZR)],
                        out_hbm.at[c, pl.ds(s * ZR, ZR)])

    return body(dst3, onehot)


@jax.jit
def _sc_step(z, src3, dst3):
    @functools.partial(
        pl.kernel,
        out_type=jax.ShapeDtypeStruct((NC, NPAD, C_DIM), jnp.float32),
        mesh=_mesh,
        scratch_types=[
            pltpu.VMEM((NCH, CH), jnp.int32),
            pltpu.VMEM((NCH, CH), jnp.int32),
            pltpu.VMEM((CH, C_DIM), jnp.float32),
            pltpu.VMEM_SHARED((NPAD, C_DIM), jnp.float32),
            pltpu.SemaphoreType.DMA,
        ],
        compiler_params=_sc_params,
    )
    def body(z_hbm, src_hbm, dst_hbm, out_hbm, src_v, dst_v, rows_v, acc_sh,
             sem):
        c = lax.axis_index("c")
        s = lax.axis_index("s")
        w = c * NS + s
        init = pltpu.async_copy(z_hbm.at[pl.ds(s * ZR, ZR)],
                                acc_sh.at[pl.ds(s * ZR, ZR)], sem)
        pltpu.sync_copy(src_hbm.at[w], src_v)
        pltpu.sync_copy(dst_hbm.at[w], dst_v)
        init.wait()
        plsc.subcore_barrier()

        @pl.loop(0, NCH)
        def _(j):
            pltpu.async_copy(z_hbm.at[src_v.at[j]], rows_v, sem).wait()
            pltpu.sync_copy(rows_v, acc_sh.at[dst_v.at[j]], add=True)

        plsc.subcore_barrier()
        pltpu.sync_copy(acc_sh.at[pl.ds(s * ZR, ZR)],
                        out_hbm.at[c, pl.ds(s * ZR, ZR)])

    return body(z, src3, dst3)


def _mlp_body(x_ref, w1_ref, b1_ref, w2_ref, b2_ref, h_ref):
    a = jnp.dot(x_ref[...], w1_ref[...],
                preferred_element_type=jnp.float32,
                precision=lax.Precision.HIGHEST)
    a = jnp.maximum(a + b1_ref[...], 0.0)
    h_ref[...] = jnp.dot(a, w2_ref[...],
                         preferred_element_type=jnp.float32,
                         precision=lax.Precision.HIGHEST) + b2_ref[...]


@jax.jit
def _tc_mlp(xp, W1, b1, W2, b2):
    blk = 1024
    return pl.pallas_call(
        _mlp_body,
        grid=(NPAD // blk,),
        in_specs=[
            pl.BlockSpec((blk, D_IN), lambda i: (i, 0)),
            pl.BlockSpec((D_IN, H_DIM), lambda i: (0, 0)),
            pl.BlockSpec((1, H_DIM), lambda i: (0, 0)),
            pl.BlockSpec((H_DIM, C_DIM), lambda i: (0, 0)),
            pl.BlockSpec((1, C_DIM), lambda i: (0, 0)),
        ],
        out_specs=pl.BlockSpec((blk, C_DIM), lambda i: (i, 0)),
        out_shape=jax.ShapeDtypeStruct((NPAD, C_DIM), jnp.float32),
    )(xp, W1, b1.reshape(1, H_DIM), W2, b2.reshape(1, C_DIM))


def _prep_body(hacc_ref, h_ref, t_ref, dinv_ref, dinv2_ref, z_ref, zacc_ref):
    deg = hacc_ref[0, :, 0:1] + hacc_ref[1, :, 0:1] + 1.0
    dinv = lax.rsqrt(deg)
    z = dinv * h_ref[...]
    dinv_ref[...] = dinv
    dinv2_ref[...] = dinv * dinv
    z_ref[...] = z
    zacc_ref[...] = t_ref[0, 0] * z


@jax.jit
def _tc_prep(hacc, h, t0):
    return pl.pallas_call(
        _prep_body,
        out_shape=(
            jax.ShapeDtypeStruct((NPAD, 1), jnp.float32),
            jax.ShapeDtypeStruct((NPAD, 1), jnp.float32),
            jax.ShapeDtypeStruct((NPAD, C_DIM), jnp.float32),
            jax.ShapeDtypeStruct((NPAD, C_DIM), jnp.float32),
        ),
    )(hacc, h, t0)


def _rescale_body(acc_ref, z_ref, zacc_ref, dinv2_ref, t_ref, znew_ref,
                  zaccnew_ref):
    sz = acc_ref[0] + acc_ref[1] - z_ref[...]
    znew = dinv2_ref[...] * sz
    znew_ref[...] = znew
    zaccnew_ref[...] = zacc_ref[...] + t_ref[0, 0] * znew


@jax.jit
def _tc_rescale(acc, z, zacc, dinv2, tk):
    return pl.pallas_call(
        _rescale_body,
        out_shape=(
            jax.ShapeDtypeStruct((NPAD, C_DIM), jnp.float32),
            jax.ShapeDtypeStruct((NPAD, C_DIM), jnp.float32),
        ),
    )(acc, z, zacc, dinv2, tk)


def _final_body(zacc_ref, dinv_ref, out_ref):
    hid = zacc_ref[...] / dinv_ref[...]
    m = jnp.max(hid, axis=1, keepdims=True)
    e = jnp.exp(hid - m)
    lse = jnp.log(jnp.sum(e, axis=1, keepdims=True))
    out_ref[...] = hid - m - lse


@jax.jit
def _tc_final(zacc, dinv):
    return pl.pallas_call(
        _final_body,
        out_shape=jax.ShapeDtypeStruct((NPAD, C_DIM), jnp.float32),
    )(zacc, dinv)


def kernel(x, edge_index, W1, b1, W2, b2, temp):
    E = edge_index.shape[1]
    pad = E_TOT - E
    pad_idx = N + (jnp.arange(pad, dtype=jnp.int32) % (NPAD - N))
    src3 = jnp.concatenate([edge_index[0], pad_idx]).reshape(NW, NCH, CH)
    dst3 = jnp.concatenate([edge_index[1], pad_idx]).reshape(NW, NCH, CH)
    onehot = jnp.zeros((CH, 16), jnp.float32).at[:, 0].set(1.0)
    xp = jnp.pad(x, ((0, NPAD - N), (0, 0)))

    hacc = _sc_degree(dst3, onehot)
    h = _tc_mlp(xp, W1, b1, W2, b2)
    tc = temp.reshape(1, K_HOPS + 1)
    dinv, dinv2, z, zacc = _tc_prep(hacc, h, tc[:, 0:1])
    for k in range(K_HOPS):
        acc = _sc_step(z, src3, dst3)
        z, zacc = _tc_rescale(acc, z, zacc, dinv2, tc[:, k + 1:k + 2])
    out = _tc_final(zacc, dinv)
    return out[:N]
